# Initial kernel scaffold; baseline (speedup 1.0000x reference)
import jax
import jax.numpy as jnp
from jax.experimental import pallas as pl


def kernel(attended_nodes, node_attention, selected_edges_l, memorized_embedding, rel_emb_l, query_src_emb, query_rel_emb, query_time_emb, W_proj, b_proj, W_ps, b_ps, W_pt, b_pt, W_left, b_left, W_right, b_right, W_center, b_center):
    raise NotImplementedError("write your pallas kernel here")


# speedup vs baseline: 7.9982x; 7.9982x over previous
import functools
import jax
import jax.numpy as jnp
from jax import lax
from jax.experimental import pallas as pl
from jax.experimental.pallas import tpu as pltpu
from jax.experimental.pallas import tpu_sc as plsc

N = 10000
E = 160000
B = 128
NPAD = 10240
NSLICE = NPAD // 16
C = 1280
GRID = E // C
EPT = E // 16
GK = 1000
NEG = -3.4e38

_f32 = jnp.float32
_i32 = jnp.int32


def _lrelu(x):
    return jnp.where(x >= 0, x, 0.01 * x)


def _prep_body(mem_ref, wproj_ref, bproj_ref, wl_ref, bl_ref, wr_ref, br_ref,
               qs_ref, wps_ref, bps_ref, qr_ref, qt_ref, wpt_ref, bpt_ref,
               nl_ref, nr_ref, qlr_ref):
    wproj = wproj_ref[...]
    wl = wl_ref[...]
    wr = wr_ref[...]
    bproj = bproj_ref[...]
    dot = functools.partial(jnp.dot, preferred_element_type=_f32)

    p = dot(mem_ref[...], wproj.T) + bproj
    nl_ref[...] = dot(p, wl[:, :64].T)
    nr_ref[...] = dot(p, wr[:, :64].T)

    qsv = dot(qs_ref[...], wps_ref[...].T) + bps_ref[...]
    qrv = dot(qr_ref[...], wproj.T) + bproj
    qtv = dot(qt_ref[...], wpt_ref[...].T) + bpt_ref[...]

    ql = (dot(qsv, wl[:, 128:160].T) + dot(qrv, wl[:, 160:224].T)
          + dot(qtv, wl[:, 224:256].T) + bl_ref[...])
    qr_out = (dot(qsv, wr[:, 128:160].T) + dot(qrv, wr[:, 160:224].T)
              + dot(qtv, wr[:, 224:256].T) + br_ref[...])
    qlr_ref[...] = jnp.concatenate([ql, qr_out], axis=1)


def _prep(mem, wproj, bproj, wl, bl, wr, br, qs, wps, bps, qr, qt, wpt, bpt):
    outs = [
        jax.ShapeDtypeStruct((N, 64), _f32),
        jax.ShapeDtypeStruct((N, 64), _f32),
        jax.ShapeDtypeStruct((B, 128), _f32),
    ]
    return pl.pallas_call(_prep_body, out_shape=outs)(
        mem, wproj, bproj, wl, bl, wr, br, qs, wps, bps, qr, qt, wpt, bpt)


def _gather_body(nl_hbm, nr_hbm, ii_hbm, ij_hbm, gl_hbm, gr_hbm,
                 idx_v, rows_v, sem):
    wid = lax.axis_index("s") * 2 + lax.axis_index("c")
    base = wid * (E // 32)
    for tab, idx_hbm, out_hbm in ((nl_hbm, ii_hbm, gl_hbm),
                                  (nr_hbm, ij_hbm, gr_hbm)):
        for ch in range(E // 32 // GK):
            off = base + ch * GK
            pltpu.sync_copy(idx_hbm.at[pl.ds(off, GK)], idx_v)
            pltpu.async_copy(tab.at[idx_v], rows_v, sem).wait()
            pltpu.sync_copy(rows_v, out_hbm.at[pl.ds(off, GK)])


def _gather(nl, nr, ii, ij):
    mesh = plsc.VectorSubcoreMesh(core_axis_name="c", subcore_axis_name="s",
                                  num_cores=2, num_subcores=16)
    fn = pl.kernel(
        _gather_body,
        out_type=[jax.ShapeDtypeStruct((E, 64), _f32),
                  jax.ShapeDtypeStruct((E, 64), _f32)],
        mesh=mesh,
        scratch_types=[pltpu.VMEM((GK,), _i32),
                       pltpu.VMEM((GK, 64), _f32),
                       pltpu.SemaphoreType.DMA],
        compiler_params=pltpu.CompilerParams(use_tc_tiling_on_sc=False),
    )
    return fn(nl, nr, ii, ij)


def _logits_body(rel_ref, gl_ref, gr_ref, eg_ref, wproj_ref, bproj_ref,
                 wl_ref, wr_ref, qlr_ref, wc_ref, bc_ref, out_ref):
    d = functools.partial(jnp.dot, preferred_element_type=_f32)
    rel_emb = d(rel_ref[...], wproj_ref[...].T) + bproj_ref[...]
    rel_l = d(rel_emb, wl_ref[...][:, 64:128].T)
    rel_r = d(rel_emb, wr_ref[...][:, 64:128].T)
    eg = eg_ref[0, 0, :]
    onehot = (eg[:, None]
              == lax.broadcasted_iota(_i32, (C, B), 1)).astype(jnp.bfloat16)
    qlr = qlr_ref[...]
    qh = qlr.astype(jnp.bfloat16)
    ql = (qlr - qh.astype(_f32)).astype(jnp.bfloat16)
    q = d(onehot, qh) + d(onehot, ql)
    pre_l = rel_l + q[:, :64] + gl_ref[...]
    pre_r = rel_r + q[:, 64:] + gr_ref[...]
    lh = _lrelu(pre_l)
    rh = d(_lrelu(pre_r), wc_ref[...].T) + bc_ref[...]
    out_ref[0, 0, :] = jnp.sum(lh * rh, axis=1)


def _logits(rel, gl, gr, eg3, wproj, bproj, wl, wr, qlr, wc, bc):
    full = lambda shape: pl.BlockSpec(shape, lambda i: (0,) * len(shape))
    return pl.pallas_call(
        _logits_body,
        grid=(GRID,),
        in_specs=[
            pl.BlockSpec((C, 256), lambda i: (i, 0)),
            pl.BlockSpec((C, 64), lambda i: (i, 0)),
            pl.BlockSpec((C, 64), lambda i: (i, 0)),
            pl.BlockSpec((1, 1, C), lambda i: (i, 0, 0)),
            full((64, 256)),
            full((1, 64)),
            full((64, 256)),
            full((64, 256)),
            full((B, 128)),
            full((64, 64)),
            full((1, 64)),
        ],
        out_specs=pl.BlockSpec((1, 1, C), lambda i: (i, 0, 0)),
        out_shape=jax.ShapeDtypeStruct((GRID, 1, C), _f32),
    )(rel, gl, gr, eg3, wproj, bproj, wl, wr, qlr, wc, bc)


def _sm_fill(ref, value):
    def body(k, _):
        ref[pl.ds(k * 16, 16)] = jnp.full((16,), value, _f32)
        return 0
    lax.fori_loop(0, NPAD // 16, body, 0)


def _softmax_body(log_hbm, ii_hbm, natt_hbm, out_hbm,
                  logv, idxv, ev, loc, mrg, tmp, stage, mst, sem):
    del sem
    c = lax.axis_index("c")
    s = lax.axis_index("s")
    base = s * EPT
    pltpu.sync_copy(log_hbm.at[pl.ds(base, EPT)], logv)
    pltpu.sync_copy(ii_hbm.at[pl.ds(base, EPT)], idxv)

    _sm_fill(loc, NEG)

    def seg_max_group(g, _):
        idx = idxv[pl.ds(g * 16, 16)]
        val = logv[pl.ds(g * 16, 16)]
        cur = plsc.load_gather(loc, [idx])
        mask = val > cur

        def cond(m):
            return jnp.any(m)

        def body(m):
            plsc.store_scatter(loc, [idx], val, mask=m)
            cur2 = plsc.load_gather(loc, [idx])
            return val > cur2

        lax.while_loop(cond, body, mask)
        return 0

    lax.fori_loop(0, EPT // 16, seg_max_group, 0)

    def merge(op):
        pltpu.sync_copy(loc, stage.at[s])
        plsc.subcore_barrier()
        col = s * NSLICE
        pltpu.sync_copy(stage.at[0, pl.ds(col, NSLICE)],
                        mrg.at[pl.ds(col, NSLICE)])
        for k in range(1, 16):
            pltpu.sync_copy(stage.at[k, pl.ds(col, NSLICE)], tmp)

            def acc(v, _):
                off = col + v * 16
                mrg[pl.ds(off, 16)] = op(mrg[pl.ds(off, 16)],
                                         tmp[pl.ds(v * 16, 16)])
                return 0

            lax.fori_loop(0, NSLICE // 16, acc, 0)
        pltpu.sync_copy(mrg.at[pl.ds(col, NSLICE)],
                        mst.at[pl.ds(col, NSLICE)])
        plsc.subcore_barrier()
        pltpu.sync_copy(mst, mrg)

    merge(jnp.maximum)

    _sm_fill(loc, 0.0)

    def exp_group(g, _):
        sl = pl.ds(g * 16, 16)
        idx = idxv[sl]
        m = plsc.load_gather(mrg, [idx])
        e = jnp.exp(logv[sl] - m)
        ev[sl] = e
        plsc.addupdate_scatter(loc, [idx], e)
        return 0

    lax.fori_loop(0, EPT // 16, exp_group, 0)

    merge(jnp.add)

    pltpu.sync_copy(natt_hbm, loc.at[pl.ds(0, N)])

    def fac(v, _):
        sl = pl.ds(v * 16, 16)
        loc[sl] = loc[sl] / mrg[sl]
        return 0

    lax.fori_loop(0, N // 16, fac, 0)

    def out_group(g, _):
        sl = pl.ds(g * 16, 16)
        ev[sl] = ev[sl] * plsc.load_gather(loc, [idxv[sl]])
        return 0

    lax.fori_loop(0, EPT // 16, out_group, 0)

    @pl.when(c == 0)
    def _():
        pltpu.sync_copy(ev, out_hbm.at[pl.ds(base, EPT)])


def _softmax(logits, ii, natt):
    mesh = plsc.VectorSubcoreMesh(core_axis_name="c", subcore_axis_name="s",
                                  num_cores=2, num_subcores=16)
    fn = pl.kernel(
        _softmax_body,
        out_type=jax.ShapeDtypeStruct((E,), _f32),
        mesh=mesh,
        scratch_types=[
            pltpu.VMEM((EPT,), _f32),
            pltpu.VMEM((EPT,), _i32),
            pltpu.VMEM((EPT,), _f32),
            pltpu.VMEM((NPAD,), _f32),
            pltpu.VMEM((NPAD,), _f32),
            pltpu.VMEM((NSLICE,), _f32),
            pltpu.VMEM_SHARED((16, NPAD), _f32),
            pltpu.VMEM_SHARED((NPAD,), _f32),
            pltpu.SemaphoreType.DMA,
        ],
        compiler_params=pltpu.CompilerParams(needs_layout_passes=False),
    )
    return fn(logits, ii, natt)


def kernel(attended_nodes, node_attention, selected_edges_l,
           memorized_embedding, rel_emb_l, query_src_emb, query_rel_emb,
           query_time_emb, W_proj, b_proj, W_ps, b_ps, W_pt, b_pt,
           W_left, b_left, W_right, b_right, W_center, b_center):
    del attended_nodes
    edges = selected_edges_l[-1]
    eg = edges[:, 0].astype(_i32)
    ii = edges[:, 6].astype(_i32)
    ij = edges[:, 7].astype(_i32)
    rel = rel_emb_l[-1]

    row = lambda v: v.reshape(1, -1)
    nl, nr, qlr = _prep(
        memorized_embedding, W_proj, row(b_proj), W_left, row(b_left),
        W_right, row(b_right), query_src_emb, W_ps, row(b_ps),
        query_rel_emb, query_time_emb, W_pt, row(b_pt))

    gl, gr = _gather(nl, nr, ii, ij)

    logits3 = _logits(rel, gl, gr, eg.reshape(GRID, 1, C), W_proj,
                      row(b_proj), W_left, W_right, qlr,
                      W_center, row(b_center))
    logits = logits3.reshape(E)

    return _softmax(logits, ii, node_attention)


# speedup vs baseline: 9.7767x; 1.2224x over previous
import functools
import jax
import jax.numpy as jnp
from jax import lax
from jax.experimental import pallas as pl
from jax.experimental.pallas import tpu as pltpu
from jax.experimental.pallas import tpu_sc as plsc

N = 10000
E = 160000
B = 128
NPAD = 10240
NSLICE = NPAD // 16
C = 1280
GRID = E // C
EPT = E // 16
GK = 1000
NEG = -3.4e38

_f32 = jnp.float32
_i32 = jnp.int32


def _lrelu(x):
    return jnp.where(x >= 0, x, 0.01 * x)


def _prep_body(mem_ref, wproj_ref, bproj_ref, wl_ref, bl_ref, wr_ref, br_ref,
               qs_ref, wps_ref, bps_ref, qr_ref, qt_ref, wpt_ref, bpt_ref,
               nl_ref, nr_ref, qlr_ref):
    wproj = wproj_ref[...]
    wl = wl_ref[...]
    wr = wr_ref[...]
    bproj = bproj_ref[...]
    dot = functools.partial(jnp.dot, preferred_element_type=_f32)

    p = dot(mem_ref[...], wproj.T) + bproj
    nl_ref[...] = dot(p, wl[:, :64].T)
    nr_ref[...] = dot(p, wr[:, :64].T)

    qsv = dot(qs_ref[...], wps_ref[...].T) + bps_ref[...]
    qrv = dot(qr_ref[...], wproj.T) + bproj
    qtv = dot(qt_ref[...], wpt_ref[...].T) + bpt_ref[...]

    ql = (dot(qsv, wl[:, 128:160].T) + dot(qrv, wl[:, 160:224].T)
          + dot(qtv, wl[:, 224:256].T) + bl_ref[...])
    qr_out = (dot(qsv, wr[:, 128:160].T) + dot(qrv, wr[:, 160:224].T)
              + dot(qtv, wr[:, 224:256].T) + br_ref[...])
    qlr_ref[...] = jnp.concatenate([ql, qr_out], axis=1)


def _prep(mem, wproj, bproj, wl, bl, wr, br, qs, wps, bps, qr, qt, wpt, bpt):
    outs = [
        jax.ShapeDtypeStruct((N, 64), _f32),
        jax.ShapeDtypeStruct((N, 64), _f32),
        jax.ShapeDtypeStruct((B, 128), _f32),
    ]
    return pl.pallas_call(_prep_body, out_shape=outs)(
        mem, wproj, bproj, wl, bl, wr, br, qs, wps, bps, qr, qt, wpt, bpt)


def _gather_body(nl_hbm, nr_hbm, ii_hbm, ij_hbm, gl_hbm, gr_hbm,
                 idx2, rows2, gs0, gs1, ws0, ws1):
    wid = lax.axis_index("s") * 2 + lax.axis_index("c")
    base = wid * (E // 32)
    nch = E // 32 // GK
    steps = []
    for tab, idx_hbm, out_hbm in ((nl_hbm, ii_hbm, gl_hbm),
                                  (nr_hbm, ij_hbm, gr_hbm)):
        for ch in range(nch):
            steps.append((tab, idx_hbm, out_hbm, base + ch * GK))
    gs = (gs0, gs1)
    ws = (ws0, ws1)
    n = len(steps)

    def start_gather(k):
        tab, ih, _, off = steps[k]
        b = k % 2
        pltpu.sync_copy(ih.at[pl.ds(off, GK)], idx2.at[b])
        return pltpu.async_copy(tab.at[idx2.at[b]], rows2.at[b], gs[b])

    gh = {0: start_gather(0)}
    wh = {}
    for k in range(n):
        b = k % 2
        if k + 1 < n:
            if k - 1 >= 0:
                wh[k - 1].wait()
            gh[k + 1] = start_gather(k + 1)
        gh[k].wait()
        _, _, oh, off = steps[k]
        wh[k] = pltpu.async_copy(rows2.at[b], oh.at[pl.ds(off, GK)], ws[b])
    wh[n - 2].wait()
    wh[n - 1].wait()


def _gather(nl, nr, ii, ij):
    mesh = plsc.VectorSubcoreMesh(core_axis_name="c", subcore_axis_name="s",
                                  num_cores=2, num_subcores=16)
    fn = pl.kernel(
        _gather_body,
        out_type=[jax.ShapeDtypeStruct((E, 64), _f32),
                  jax.ShapeDtypeStruct((E, 64), _f32)],
        mesh=mesh,
        scratch_types=[pltpu.VMEM((2, GK), _i32),
                       pltpu.VMEM((2, GK, 64), _f32),
                       pltpu.SemaphoreType.DMA,
                       pltpu.SemaphoreType.DMA,
                       pltpu.SemaphoreType.DMA,
                       pltpu.SemaphoreType.DMA],
        compiler_params=pltpu.CompilerParams(use_tc_tiling_on_sc=False),
    )
    return fn(nl, nr, ii, ij)


def _logits_body(rel_ref, gl_ref, gr_ref, eg_ref, wproj_ref, bproj_ref,
                 wl_ref, wr_ref, qlr_ref, wc_ref, bc_ref, out_ref):
    d = functools.partial(jnp.dot, preferred_element_type=_f32)
    rel_emb = d(rel_ref[...], wproj_ref[...].T) + bproj_ref[...]
    rel_l = d(rel_emb, wl_ref[...][:, 64:128].T)
    rel_r = d(rel_emb, wr_ref[...][:, 64:128].T)
    eg = eg_ref[0, 0, :]
    onehot = (eg[:, None]
              == lax.broadcasted_iota(_i32, (C, B), 1)).astype(jnp.bfloat16)
    qlr = qlr_ref[...]
    qh = qlr.astype(jnp.bfloat16)
    ql = (qlr - qh.astype(_f32)).astype(jnp.bfloat16)
    q = d(onehot, qh) + d(onehot, ql)
    pre_l = rel_l + q[:, :64] + gl_ref[...]
    pre_r = rel_r + q[:, 64:] + gr_ref[...]
    lh = _lrelu(pre_l)
    rh = d(_lrelu(pre_r), wc_ref[...].T) + bc_ref[...]
    out_ref[0, 0, :] = jnp.sum((lh * rh).T, axis=0)


def _logits(rel, gl, gr, eg3, wproj, bproj, wl, wr, qlr, wc, bc):
    full = lambda shape: pl.BlockSpec(shape, lambda i: (0,) * len(shape))
    return pl.pallas_call(
        _logits_body,
        grid=(GRID,),
        in_specs=[
            pl.BlockSpec((C, 256), lambda i: (i, 0)),
            pl.BlockSpec((C, 64), lambda i: (i, 0)),
            pl.BlockSpec((C, 64), lambda i: (i, 0)),
            pl.BlockSpec((1, 1, C), lambda i: (i, 0, 0)),
            full((64, 256)),
            full((1, 64)),
            full((64, 256)),
            full((64, 256)),
            full((B, 128)),
            full((64, 64)),
            full((1, 64)),
        ],
        out_specs=pl.BlockSpec((1, 1, C), lambda i: (i, 0, 0)),
        out_shape=jax.ShapeDtypeStruct((GRID, 1, C), _f32),
    )(rel, gl, gr, eg3, wproj, bproj, wl, wr, qlr, wc, bc)


def _sm_fill(ref, value):
    def body(k, _):
        ref[pl.ds(k * 16, 16)] = jnp.full((16,), value, _f32)
        return 0
    lax.fori_loop(0, NPAD // 16, body, 0)


def _softmax_body(log_hbm, ii_hbm, natt_hbm, out_hbm,
                  logv, idxv, ev, loc, mrg, tmp, stage, mst, sem):
    del sem
    c = lax.axis_index("c")
    s = lax.axis_index("s")
    base = s * EPT
    pltpu.sync_copy(log_hbm.at[pl.ds(base, EPT)], logv)
    pltpu.sync_copy(ii_hbm.at[pl.ds(base, EPT)], idxv)

    _sm_fill(loc, NEG)

    def seg_max_group(g, _):
        idx = idxv[pl.ds(g * 16, 16)]
        val = logv[pl.ds(g * 16, 16)]
        cur = plsc.load_gather(loc, [idx])
        mask = val > cur

        def cond(m):
            return jnp.any(m)

        def body(m):
            plsc.store_scatter(loc, [idx], val, mask=m)
            cur2 = plsc.load_gather(loc, [idx])
            return val > cur2

        lax.while_loop(cond, body, mask)
        return 0

    lax.fori_loop(0, EPT // 16, seg_max_group, 0)

    def merge(op):
        pltpu.sync_copy(loc, stage.at[s])
        plsc.subcore_barrier()
        col = s * NSLICE
        pltpu.sync_copy(stage.at[0, pl.ds(col, NSLICE)],
                        mrg.at[pl.ds(col, NSLICE)])
        for k in range(1, 16):
            pltpu.sync_copy(stage.at[k, pl.ds(col, NSLICE)], tmp)

            def acc(v, _):
                off = col + v * 16
                mrg[pl.ds(off, 16)] = op(mrg[pl.ds(off, 16)],
                                         tmp[pl.ds(v * 16, 16)])
                return 0

            lax.fori_loop(0, NSLICE // 16, acc, 0)
        pltpu.sync_copy(mrg.at[pl.ds(col, NSLICE)],
                        mst.at[pl.ds(col, NSLICE)])
        plsc.subcore_barrier()
        pltpu.sync_copy(mst, mrg)

    merge(jnp.maximum)

    _sm_fill(loc, 0.0)

    def exp_group(g, _):
        sl = pl.ds(g * 16, 16)
        idx = idxv[sl]
        m = plsc.load_gather(mrg, [idx])
        e = jnp.exp(logv[sl] - m)
        ev[sl] = e
        plsc.addupdate_scatter(loc, [idx], e)
        return 0

    lax.fori_loop(0, EPT // 16, exp_group, 0)

    merge(jnp.add)

    pltpu.sync_copy(natt_hbm, loc.at[pl.ds(0, N)])

    def fac(v, _):
        sl = pl.ds(v * 16, 16)
        loc[sl] = loc[sl] / mrg[sl]
        return 0

    lax.fori_loop(0, N // 16, fac, 0)

    def out_group(g, _):
        sl = pl.ds(g * 16, 16)
        ev[sl] = ev[sl] * plsc.load_gather(loc, [idxv[sl]])
        return 0

    lax.fori_loop(0, EPT // 16, out_group, 0)

    @pl.when(c == 0)
    def _():
        pltpu.sync_copy(ev, out_hbm.at[pl.ds(base, EPT)])


def _softmax(logits, ii, natt):
    mesh = plsc.VectorSubcoreMesh(core_axis_name="c", subcore_axis_name="s",
                                  num_cores=2, num_subcores=16)
    fn = pl.kernel(
        _softmax_body,
        out_type=jax.ShapeDtypeStruct((E,), _f32),
        mesh=mesh,
        scratch_types=[
            pltpu.VMEM((EPT,), _f32),
            pltpu.VMEM((EPT,), _i32),
            pltpu.VMEM((EPT,), _f32),
            pltpu.VMEM((NPAD,), _f32),
            pltpu.VMEM((NPAD,), _f32),
            pltpu.VMEM((NSLICE,), _f32),
            pltpu.VMEM_SHARED((16, NPAD), _f32),
            pltpu.VMEM_SHARED((NPAD,), _f32),
            pltpu.SemaphoreType.DMA,
        ],
        compiler_params=pltpu.CompilerParams(needs_layout_passes=False),
    )
    return fn(logits, ii, natt)


def kernel(attended_nodes, node_attention, selected_edges_l,
           memorized_embedding, rel_emb_l, query_src_emb, query_rel_emb,
           query_time_emb, W_proj, b_proj, W_ps, b_ps, W_pt, b_pt,
           W_left, b_left, W_right, b_right, W_center, b_center):
    del attended_nodes
    edges = selected_edges_l[-1]
    eg = edges[:, 0].astype(_i32)
    ii = edges[:, 6].astype(_i32)
    ij = edges[:, 7].astype(_i32)
    rel = rel_emb_l[-1]

    row = lambda v: v.reshape(1, -1)
    nl, nr, qlr = _prep(
        memorized_embedding, W_proj, row(b_proj), W_left, row(b_left),
        W_right, row(b_right), query_src_emb, W_ps, row(b_ps),
        query_rel_emb, query_time_emb, W_pt, row(b_pt))

    gl, gr = _gather(nl, nr, ii, ij)

    logits3 = _logits(rel, gl, gr, eg.reshape(GRID, 1, C), W_proj,
                      row(b_proj), W_left, W_right, qlr,
                      W_center, row(b_center))
    logits = logits3.reshape(E)

    return _softmax(logits, ii, node_attention)


# speedup vs baseline: 13.9095x; 1.4227x over previous
import functools
import jax
import jax.numpy as jnp
from jax import lax
from jax.experimental import pallas as pl
from jax.experimental.pallas import tpu as pltpu
from jax.experimental.pallas import tpu_sc as plsc

N = 10000
E = 160000
B = 128
NPAD = 10240
NSLICE = NPAD // 16
C = 1280
GRID = E // C
EPT = E // 16
GK = 1000
NEG = -3.4e38

_f32 = jnp.float32
_i32 = jnp.int32


def _lrelu(x):
    return jnp.where(x >= 0, x, 0.01 * x)


def _prep_body(mem_ref, wproj_ref, bproj_ref, wl_ref, bl_ref, wr_ref, br_ref,
               qs_ref, wps_ref, bps_ref, qr_ref, qt_ref, wpt_ref, bpt_ref,
               nl_ref, nr_ref, qlr_ref):
    wproj = wproj_ref[...]
    wl = wl_ref[...]
    wr = wr_ref[...]
    bproj = bproj_ref[...]
    dot = functools.partial(jnp.dot, preferred_element_type=_f32)

    p = dot(mem_ref[...], wproj.T) + bproj
    nl_ref[...] = dot(p, wl[:, :64].T)
    nr_ref[...] = dot(p, wr[:, :64].T)

    qsv = dot(qs_ref[...], wps_ref[...].T) + bps_ref[...]
    qrv = dot(qr_ref[...], wproj.T) + bproj
    qtv = dot(qt_ref[...], wpt_ref[...].T) + bpt_ref[...]

    ql = (dot(qsv, wl[:, 128:160].T) + dot(qrv, wl[:, 160:224].T)
          + dot(qtv, wl[:, 224:256].T) + bl_ref[...])
    qr_out = (dot(qsv, wr[:, 128:160].T) + dot(qrv, wr[:, 160:224].T)
              + dot(qtv, wr[:, 224:256].T) + br_ref[...])
    qlr_ref[...] = jnp.concatenate([ql, qr_out], axis=1)


def _prep(mem, wproj, bproj, wl, bl, wr, br, qs, wps, bps, qr, qt, wpt, bpt):
    outs = [
        jax.ShapeDtypeStruct((N, 64), _f32),
        jax.ShapeDtypeStruct((N, 64), _f32),
        jax.ShapeDtypeStruct((B, 128), _f32),
    ]
    return pl.pallas_call(_prep_body, out_shape=outs)(
        mem, wproj, bproj, wl, bl, wr, br, qs, wps, bps, qr, qt, wpt, bpt)


def _gather_body(nl_hbm, nr_hbm, ii_hbm, ij_hbm, glr_hbm,
                 idx2, rows2, gs0, gs1, ws0, ws1):
    wid = lax.axis_index("s") * 2 + lax.axis_index("c")
    base = wid * (E // 32)
    nch = E // 32 // GK
    steps = []
    for col, (tab, idx_hbm) in enumerate(((nl_hbm, ii_hbm), (nr_hbm, ij_hbm))):
        for ch in range(nch):
            steps.append((tab, idx_hbm, col * 64, base + ch * GK))
    gs = (gs0, gs1)
    ws = (ws0, ws1)
    n = len(steps)

    def start_gather(k):
        tab, ih, _, off = steps[k]
        b = k % 2
        pltpu.sync_copy(ih.at[pl.ds(off, GK)], idx2.at[b])
        return pltpu.async_copy(tab.at[idx2.at[b]], rows2.at[b], gs[b])

    gh = {0: start_gather(0)}
    wh = {}
    for k in range(n):
        b = k % 2
        if k + 1 < n:
            if k - 1 >= 0:
                wh[k - 1].wait()
            gh[k + 1] = start_gather(k + 1)
        gh[k].wait()
        _, _, col, off = steps[k]
        wh[k] = pltpu.async_copy(
            rows2.at[b], glr_hbm.at[pl.ds(off, GK), pl.ds(col, 64)], ws[b])
    wh[n - 2].wait()
    wh[n - 1].wait()


def _gather(nl, nr, ii, ij):
    mesh = plsc.VectorSubcoreMesh(core_axis_name="c", subcore_axis_name="s",
                                  num_cores=2, num_subcores=16)
    fn = pl.kernel(
        _gather_body,
        out_type=jax.ShapeDtypeStruct((E, 128), _f32),
        mesh=mesh,
        scratch_types=[pltpu.VMEM((2, GK), _i32),
                       pltpu.VMEM((2, GK, 64), _f32),
                       pltpu.SemaphoreType.DMA,
                       pltpu.SemaphoreType.DMA,
                       pltpu.SemaphoreType.DMA,
                       pltpu.SemaphoreType.DMA],
        compiler_params=pltpu.CompilerParams(use_tc_tiling_on_sc=False),
    )
    return fn(nl, nr, ii, ij)


def _logits_body(rel_ref, glr_ref, eg_ref, wproj_ref, bproj_ref,
                 wl_ref, wr_ref, qlr_ref, wc_ref, bc_ref, out_ref):
    d = functools.partial(jnp.dot, preferred_element_type=_f32)
    rel_emb = d(rel_ref[...], wproj_ref[...].T) + bproj_ref[...]
    rel_l = d(rel_emb, wl_ref[...][:, 64:128].T)
    rel_r = d(rel_emb, wr_ref[...][:, 64:128].T)
    eg = eg_ref[0, 0, :]
    onehot = (eg[:, None]
              == lax.broadcasted_iota(_i32, (C, B), 1)).astype(jnp.bfloat16)
    qlr = qlr_ref[...]
    qh = qlr.astype(jnp.bfloat16)
    ql = (qlr - qh.astype(_f32)).astype(jnp.bfloat16)
    q = d(onehot, qh) + d(onehot, ql)
    glr = glr_ref[...]
    pre_l = rel_l + q[:, :64] + glr[:, :64]
    pre_r = rel_r + q[:, 64:] + glr[:, 64:]
    lh = _lrelu(pre_l)
    rh = d(_lrelu(pre_r), wc_ref[...].T) + bc_ref[...]
    out_ref[0, 0, :] = jnp.sum((lh * rh).T, axis=0)


def _logits(rel, glr, eg3, wproj, bproj, wl, wr, qlr, wc, bc):
    full = lambda shape: pl.BlockSpec(shape, lambda i: (0,) * len(shape))
    return pl.pallas_call(
        _logits_body,
        grid=(GRID,),
        in_specs=[
            pl.BlockSpec((C, 256), lambda i: (i, 0)),
            pl.BlockSpec((C, 128), lambda i: (i, 0)),
            pl.BlockSpec((1, 1, C), lambda i: (i, 0, 0)),
            full((64, 256)),
            full((1, 64)),
            full((64, 256)),
            full((64, 256)),
            full((B, 128)),
            full((64, 64)),
            full((1, 64)),
        ],
        out_specs=pl.BlockSpec((1, 1, C), lambda i: (i, 0, 0)),
        out_shape=jax.ShapeDtypeStruct((GRID, 1, C), _f32),
    )(rel, glr, eg3, wproj, bproj, wl, wr, qlr, wc, bc)


def _sm_fill(ref, value):
    def body(k, _):
        ref[pl.ds(k * 16, 16)] = jnp.full((16,), value, _f32)
        return 0
    lax.fori_loop(0, NPAD // 16, body, 0)


def _softmax_body(log_hbm, ii_hbm, natt_hbm, out_hbm,
                  logv, idxv, ev, loc, mrg, tmp, stage, mst, sem):
    del sem
    c = lax.axis_index("c")
    s = lax.axis_index("s")
    base = s * EPT
    pltpu.sync_copy(log_hbm.at[pl.ds(base, EPT)], logv)
    pltpu.sync_copy(ii_hbm.at[pl.ds(base, EPT)], idxv)

    _sm_fill(loc, NEG)

    def seg_max_group(g, _):
        idx = idxv[pl.ds(g * 16, 16)]
        val = logv[pl.ds(g * 16, 16)]
        cur = plsc.load_gather(loc, [idx])
        mask = val > cur

        def cond(m):
            return jnp.any(m)

        def body(m):
            plsc.store_scatter(loc, [idx], val, mask=m)
            cur2 = plsc.load_gather(loc, [idx])
            return val > cur2

        lax.while_loop(cond, body, mask)
        return 0

    lax.fori_loop(0, EPT // 16, seg_max_group, 0)

    def merge(op):
        pltpu.sync_copy(loc, stage.at[s])
        plsc.subcore_barrier()
        col = s * NSLICE
        pltpu.sync_copy(stage.at[:, pl.ds(col, NSLICE)], tmp)

        def acc(v, _):
            sl = pl.ds(v * 16, 16)
            x = tmp[0, sl]
            for k in range(1, 16):
                x = op(x, tmp[k, sl])
            mrg[pl.ds(col + v * 16, 16)] = x
            return 0

        lax.fori_loop(0, NSLICE // 16, acc, 0)
        pltpu.sync_copy(mrg.at[pl.ds(col, NSLICE)],
                        mst.at[pl.ds(col, NSLICE)])
        plsc.subcore_barrier()
        pltpu.sync_copy(mst, mrg)

    merge(jnp.maximum)

    _sm_fill(loc, 0.0)

    def exp_group(g, _):
        sl = pl.ds(g * 16, 16)
        idx = idxv[sl]
        m = plsc.load_gather(mrg, [idx])
        e = jnp.exp(logv[sl] - m)
        ev[sl] = e
        plsc.addupdate_scatter(loc, [idx], e)
        return 0

    lax.fori_loop(0, EPT // 16, exp_group, 0)

    merge(jnp.add)

    pltpu.sync_copy(natt_hbm, loc.at[pl.ds(0, N)])

    def fac(v, _):
        sl = pl.ds(v * 16, 16)
        loc[sl] = loc[sl] / mrg[sl]
        return 0

    lax.fori_loop(0, N // 16, fac, 0)

    def out_group(g, _):
        sl = pl.ds(g * 16, 16)
        ev[sl] = ev[sl] * plsc.load_gather(loc, [idxv[sl]])
        return 0

    lax.fori_loop(0, EPT // 16, out_group, 0)

    @pl.when(c == 0)
    def _():
        pltpu.sync_copy(ev, out_hbm.at[pl.ds(base, EPT)])


def _softmax(logits, ii, natt):
    mesh = plsc.VectorSubcoreMesh(core_axis_name="c", subcore_axis_name="s",
                                  num_cores=2, num_subcores=16)
    fn = pl.kernel(
        _softmax_body,
        out_type=jax.ShapeDtypeStruct((E,), _f32),
        mesh=mesh,
        scratch_types=[
            pltpu.VMEM((EPT,), _f32),
            pltpu.VMEM((EPT,), _i32),
            pltpu.VMEM((EPT,), _f32),
            pltpu.VMEM((NPAD,), _f32),
            pltpu.VMEM((NPAD,), _f32),
            pltpu.VMEM((16, NSLICE), _f32),
            pltpu.VMEM_SHARED((16, NPAD), _f32),
            pltpu.VMEM_SHARED((NPAD,), _f32),
            pltpu.SemaphoreType.DMA,
        ],
        compiler_params=pltpu.CompilerParams(needs_layout_passes=False),
    )
    return fn(logits, ii, natt)


def kernel(attended_nodes, node_attention, selected_edges_l,
           memorized_embedding, rel_emb_l, query_src_emb, query_rel_emb,
           query_time_emb, W_proj, b_proj, W_ps, b_ps, W_pt, b_pt,
           W_left, b_left, W_right, b_right, W_center, b_center):
    del attended_nodes
    edges = selected_edges_l[-1]
    eg = edges[:, 0].astype(_i32)
    ii = edges[:, 6].astype(_i32)
    ij = edges[:, 7].astype(_i32)
    rel = rel_emb_l[-1]

    row = lambda v: v.reshape(1, -1)
    nl, nr, qlr = _prep(
        memorized_embedding, W_proj, row(b_proj), W_left, row(b_left),
        W_right, row(b_right), query_src_emb, W_ps, row(b_ps),
        query_rel_emb, query_time_emb, W_pt, row(b_pt))

    glr = _gather(nl, nr, ii, ij)

    logits3 = _logits(rel, glr, eg.reshape(GRID, 1, C), W_proj,
                      row(b_proj), W_left, W_right, qlr,
                      W_center, row(b_center))
    logits = logits3.reshape(E)

    return _softmax(logits, ii, node_attention)


# speedup vs baseline: 16.1802x; 1.1633x over previous
import functools
import jax
import jax.numpy as jnp
from jax import lax
from jax.experimental import pallas as pl
from jax.experimental.pallas import tpu as pltpu
from jax.experimental.pallas import tpu_sc as plsc

N = 10000
E = 160000
B = 128
NPAD = 10240
NSLICE = NPAD // 16
C = 3200
GRID = E // C
EPT = E // 16
GK = 1000
NEG = -3.4e38

_f32 = jnp.float32
_i32 = jnp.int32


def _lrelu(x):
    return jnp.where(x >= 0, x, 0.01 * x)


def _prep_body(mem_ref, wproj_ref, bproj_ref, wl_ref, bl_ref, wr_ref, br_ref,
               qs_ref, wps_ref, bps_ref, qr_ref, qt_ref, wpt_ref, bpt_ref,
               nl_ref, nr_ref, qlr_ref):
    wproj = wproj_ref[...]
    wl = wl_ref[...]
    wr = wr_ref[...]
    bproj = bproj_ref[...]
    dot = functools.partial(jnp.dot, preferred_element_type=_f32)

    p = dot(mem_ref[...], wproj.T) + bproj
    nl_ref[...] = dot(p, wl[:, :64].T)
    nr_ref[...] = dot(p, wr[:, :64].T)

    qsv = dot(qs_ref[...], wps_ref[...].T) + bps_ref[...]
    qrv = dot(qr_ref[...], wproj.T) + bproj
    qtv = dot(qt_ref[...], wpt_ref[...].T) + bpt_ref[...]

    ql = (dot(qsv, wl[:, 128:160].T) + dot(qrv, wl[:, 160:224].T)
          + dot(qtv, wl[:, 224:256].T) + bl_ref[...])
    qr_out = (dot(qsv, wr[:, 128:160].T) + dot(qrv, wr[:, 160:224].T)
              + dot(qtv, wr[:, 224:256].T) + br_ref[...])
    qlr_ref[...] = jnp.concatenate([ql, qr_out], axis=1)


def _prep(mem, wproj, bproj, wl, bl, wr, br, qs, wps, bps, qr, qt, wpt, bpt):
    outs = [
        jax.ShapeDtypeStruct((N, 64), _f32),
        jax.ShapeDtypeStruct((N, 64), _f32),
        jax.ShapeDtypeStruct((B, 128), _f32),
    ]
    return pl.pallas_call(_prep_body, out_shape=outs)(
        mem, wproj, bproj, wl, bl, wr, br, qs, wps, bps, qr, qt, wpt, bpt)


def _gather_body(nl_hbm, nr_hbm, ii_hbm, ij_hbm, glr_hbm,
                 idx2, rows2, gs0, gs1, ws0, ws1):
    wid = lax.axis_index("s") * 2 + lax.axis_index("c")
    base = wid * (E // 32)
    nch = E // 32 // GK
    steps = []
    for col, (tab, idx_hbm) in enumerate(((nl_hbm, ii_hbm), (nr_hbm, ij_hbm))):
        for ch in range(nch):
            steps.append((tab, idx_hbm, col * 64, base + ch * GK))
    gs = (gs0, gs1)
    ws = (ws0, ws1)
    n = len(steps)

    def start_gather(k):
        tab, ih, _, off = steps[k]
        b = k % 2
        pltpu.sync_copy(ih.at[pl.ds(off, GK)], idx2.at[b])
        return pltpu.async_copy(tab.at[idx2.at[b]], rows2.at[b], gs[b])

    gh = {0: start_gather(0)}
    wh = {}
    for k in range(n):
        b = k % 2
        if k + 1 < n:
            if k - 1 >= 0:
                wh[k - 1].wait()
            gh[k + 1] = start_gather(k + 1)
        gh[k].wait()
        _, _, col, off = steps[k]
        wh[k] = pltpu.async_copy(
            rows2.at[b], glr_hbm.at[pl.ds(off, GK), pl.ds(col, 64)], ws[b])
    wh[n - 2].wait()
    wh[n - 1].wait()


def _gather(nl, nr, ii, ij):
    mesh = plsc.VectorSubcoreMesh(core_axis_name="c", subcore_axis_name="s",
                                  num_cores=2, num_subcores=16)
    fn = pl.kernel(
        _gather_body,
        out_type=jax.ShapeDtypeStruct((E, 128), _f32),
        mesh=mesh,
        scratch_types=[pltpu.VMEM((2, GK), _i32),
                       pltpu.VMEM((2, GK, 64), _f32),
                       pltpu.SemaphoreType.DMA,
                       pltpu.SemaphoreType.DMA,
                       pltpu.SemaphoreType.DMA,
                       pltpu.SemaphoreType.DMA],
        compiler_params=pltpu.CompilerParams(use_tc_tiling_on_sc=False),
    )
    return fn(nl, nr, ii, ij)


def _logits_body(rel_ref, glr_ref, eg_ref, wproj_ref, bproj_ref,
                 wl_ref, wr_ref, qlr_ref, wc_ref, bc_ref, out_ref):
    d = functools.partial(jnp.dot, preferred_element_type=_f32)
    rel_emb = d(rel_ref[...], wproj_ref[...].T) + bproj_ref[...]
    rel_l = d(rel_emb, wl_ref[...][:, 64:128].T)
    rel_r = d(rel_emb, wr_ref[...][:, 64:128].T)
    eg = eg_ref[0, 0, :]
    onehot = (eg[:, None]
              == lax.broadcasted_iota(_i32, (C, B), 1)).astype(jnp.bfloat16)
    qlr = qlr_ref[...]
    qh = qlr.astype(jnp.bfloat16)
    ql = (qlr - qh.astype(_f32)).astype(jnp.bfloat16)
    q = d(onehot, qh) + d(onehot, ql)
    glr = glr_ref[...]
    pre_l = rel_l + q[:, :64] + glr[:, :64]
    pre_r = rel_r + q[:, 64:] + glr[:, 64:]
    lh = _lrelu(pre_l)
    rh = d(_lrelu(pre_r), wc_ref[...].T) + bc_ref[...]
    out_ref[0, 0, :] = jnp.sum((lh * rh).T, axis=0)


def _logits(rel, glr, eg3, wproj, bproj, wl, wr, qlr, wc, bc):
    full = lambda shape: pl.BlockSpec(shape, lambda i: (0,) * len(shape))
    return pl.pallas_call(
        _logits_body,
        grid=(GRID,),
        in_specs=[
            pl.BlockSpec((C, 256), lambda i: (i, 0)),
            pl.BlockSpec((C, 128), lambda i: (i, 0)),
            pl.BlockSpec((1, 1, C), lambda i: (i, 0, 0)),
            full((64, 256)),
            full((1, 64)),
            full((64, 256)),
            full((64, 256)),
            full((B, 128)),
            full((64, 64)),
            full((1, 64)),
        ],
        out_specs=pl.BlockSpec((1, 1, C), lambda i: (i, 0, 0)),
        out_shape=jax.ShapeDtypeStruct((GRID, 1, C), _f32),
    )(rel, glr, eg3, wproj, bproj, wl, wr, qlr, wc, bc)


def _sm_fill(ref, value):
    def body(k, _):
        ref[pl.ds(k * 16, 16)] = jnp.full((16,), value, _f32)
        return 0
    lax.fori_loop(0, NPAD // 16, body, 0)


def _softmax_body(log_hbm, ii_hbm, natt_hbm, out_hbm,
                  logv, idxv, ev, loc, mrg, tmp, stage, mst, sem):
    del sem
    c = lax.axis_index("c")
    s = lax.axis_index("s")
    base = s * EPT
    pltpu.sync_copy(log_hbm.at[pl.ds(base, EPT)], logv)
    pltpu.sync_copy(ii_hbm.at[pl.ds(base, EPT)], idxv)

    _sm_fill(loc, NEG)

    def seg_max_group(g, _):
        idx = idxv[pl.ds(g * 16, 16)]
        val = logv[pl.ds(g * 16, 16)]
        cur = plsc.load_gather(loc, [idx])
        mask = val > cur

        def cond(m):
            return jnp.any(m)

        def body(m):
            plsc.store_scatter(loc, [idx], val, mask=m)
            cur2 = plsc.load_gather(loc, [idx])
            return val > cur2

        lax.while_loop(cond, body, mask)
        return 0

    lax.fori_loop(0, EPT // 16, seg_max_group, 0)

    def merge(op):
        pltpu.sync_copy(loc, stage.at[s])
        plsc.subcore_barrier()
        col = s * NSLICE
        pltpu.sync_copy(stage.at[:, pl.ds(col, NSLICE)], tmp)

        def acc(v, _):
            sl = pl.ds(v * 16, 16)
            x = tmp[0, sl]
            for k in range(1, 16):
                x = op(x, tmp[k, sl])
            mrg[pl.ds(col + v * 16, 16)] = x
            return 0

        lax.fori_loop(0, NSLICE // 16, acc, 0)
        pltpu.sync_copy(mrg.at[pl.ds(col, NSLICE)],
                        mst.at[pl.ds(col, NSLICE)])
        plsc.subcore_barrier()
        pltpu.sync_copy(mst, mrg)

    merge(jnp.maximum)

    _sm_fill(loc, 0.0)

    def exp_group(g, _):
        sl = pl.ds(g * 16, 16)
        idx = idxv[sl]
        m = plsc.load_gather(mrg, [idx])
        e = jnp.exp(logv[sl] - m)
        ev[sl] = e
        plsc.addupdate_scatter(loc, [idx], e)
        return 0

    lax.fori_loop(0, EPT // 16, exp_group, 0)

    merge(jnp.add)

    pltpu.sync_copy(natt_hbm, loc.at[pl.ds(0, N)])

    def fac(v, _):
        sl = pl.ds(v * 16, 16)
        loc[sl] = loc[sl] / mrg[sl]
        return 0

    lax.fori_loop(0, N // 16, fac, 0)

    def out_group(g, _):
        sl = pl.ds(g * 16, 16)
        ev[sl] = ev[sl] * plsc.load_gather(loc, [idxv[sl]])
        return 0

    lax.fori_loop(0, EPT // 16, out_group, 0)

    @pl.when(c == 0)
    def _():
        pltpu.sync_copy(ev, out_hbm.at[pl.ds(base, EPT)])


def _softmax(logits, ii, natt):
    mesh = plsc.VectorSubcoreMesh(core_axis_name="c", subcore_axis_name="s",
                                  num_cores=2, num_subcores=16)
    fn = pl.kernel(
        _softmax_body,
        out_type=jax.ShapeDtypeStruct((E,), _f32),
        mesh=mesh,
        scratch_types=[
            pltpu.VMEM((EPT,), _f32),
            pltpu.VMEM((EPT,), _i32),
            pltpu.VMEM((EPT,), _f32),
            pltpu.VMEM((NPAD,), _f32),
            pltpu.VMEM((NPAD,), _f32),
            pltpu.VMEM((16, NSLICE), _f32),
            pltpu.VMEM_SHARED((16, NPAD), _f32),
            pltpu.VMEM_SHARED((NPAD,), _f32),
            pltpu.SemaphoreType.DMA,
        ],
        compiler_params=pltpu.CompilerParams(needs_layout_passes=False),
    )
    return fn(logits, ii, natt)


def kernel(attended_nodes, node_attention, selected_edges_l,
           memorized_embedding, rel_emb_l, query_src_emb, query_rel_emb,
           query_time_emb, W_proj, b_proj, W_ps, b_ps, W_pt, b_pt,
           W_left, b_left, W_right, b_right, W_center, b_center):
    del attended_nodes
    edges = selected_edges_l[-1]
    eg = edges[:, 0].astype(_i32)
    ii = edges[:, 6].astype(_i32)
    ij = edges[:, 7].astype(_i32)
    rel = rel_emb_l[-1]

    row = lambda v: v.reshape(1, -1)
    nl, nr, qlr = _prep(
        memorized_embedding, W_proj, row(b_proj), W_left, row(b_left),
        W_right, row(b_right), query_src_emb, W_ps, row(b_ps),
        query_rel_emb, query_time_emb, W_pt, row(b_pt))

    glr = _gather(nl, nr, ii, ij)

    logits3 = _logits(rel, glr, eg.reshape(GRID, 1, C), W_proj,
                      row(b_proj), W_left, W_right, qlr,
                      W_center, row(b_center))
    logits = logits3.reshape(E)

    return _softmax(logits, ii, node_attention)


# speedup vs baseline: 16.4595x; 1.0173x over previous
import functools
import jax
import jax.numpy as jnp
from jax import lax
from jax.experimental import pallas as pl
from jax.experimental.pallas import tpu as pltpu
from jax.experimental.pallas import tpu_sc as plsc

N = 10000
E = 160000
B = 128
NPAD = 10240
NSLICE = NPAD // 16
C = 6400
GRID = E // C
EPT = E // 16
GK = 1000
NEG = -3.4e38

_f32 = jnp.float32
_i32 = jnp.int32


def _lrelu(x):
    return jnp.where(x >= 0, x, 0.01 * x)


def _prep_body(mem_ref, wproj_ref, bproj_ref, wl_ref, bl_ref, wr_ref, br_ref,
               qs_ref, wps_ref, bps_ref, qr_ref, qt_ref, wpt_ref, bpt_ref,
               nl_ref, nr_ref, qlr_ref):
    wproj = wproj_ref[...]
    wl = wl_ref[...]
    wr = wr_ref[...]
    bproj = bproj_ref[...]
    dot = functools.partial(jnp.dot, preferred_element_type=_f32)

    p = dot(mem_ref[...], wproj.T) + bproj
    nl_ref[...] = dot(p, wl[:, :64].T)
    nr_ref[...] = dot(p, wr[:, :64].T)

    qsv = dot(qs_ref[...], wps_ref[...].T) + bps_ref[...]
    qrv = dot(qr_ref[...], wproj.T) + bproj
    qtv = dot(qt_ref[...], wpt_ref[...].T) + bpt_ref[...]

    ql = (dot(qsv, wl[:, 128:160].T) + dot(qrv, wl[:, 160:224].T)
          + dot(qtv, wl[:, 224:256].T) + bl_ref[...])
    qr_out = (dot(qsv, wr[:, 128:160].T) + dot(qrv, wr[:, 160:224].T)
              + dot(qtv, wr[:, 224:256].T) + br_ref[...])
    qlr_ref[...] = jnp.concatenate([ql, qr_out], axis=1)


def _prep(mem, wproj, bproj, wl, bl, wr, br, qs, wps, bps, qr, qt, wpt, bpt):
    outs = [
        jax.ShapeDtypeStruct((N, 64), _f32),
        jax.ShapeDtypeStruct((N, 64), _f32),
        jax.ShapeDtypeStruct((B, 128), _f32),
    ]
    return pl.pallas_call(_prep_body, out_shape=outs)(
        mem, wproj, bproj, wl, bl, wr, br, qs, wps, bps, qr, qt, wpt, bpt)


def _gather_body(nl_hbm, nr_hbm, ii_hbm, ij_hbm, glr_hbm,
                 idx2, rows2, gs0, gs1, ws0, ws1):
    wid = lax.axis_index("s") * 2 + lax.axis_index("c")
    base = wid * (E // 32)
    nch = E // 32 // GK
    steps = []
    for col, (tab, idx_hbm) in enumerate(((nl_hbm, ii_hbm), (nr_hbm, ij_hbm))):
        for ch in range(nch):
            steps.append((tab, idx_hbm, col * 64, base + ch * GK))
    gs = (gs0, gs1)
    ws = (ws0, ws1)
    n = len(steps)

    def start_gather(k):
        tab, ih, _, off = steps[k]
        b = k % 2
        pltpu.sync_copy(ih.at[pl.ds(off, GK)], idx2.at[b])
        return pltpu.async_copy(tab.at[idx2.at[b]], rows2.at[b], gs[b])

    gh = {0: start_gather(0)}
    wh = {}
    for k in range(n):
        b = k % 2
        if k + 1 < n:
            if k - 1 >= 0:
                wh[k - 1].wait()
            gh[k + 1] = start_gather(k + 1)
        gh[k].wait()
        _, _, col, off = steps[k]
        wh[k] = pltpu.async_copy(
            rows2.at[b], glr_hbm.at[pl.ds(off, GK), pl.ds(col, 64)], ws[b])
    wh[n - 2].wait()
    wh[n - 1].wait()


def _gather(nl, nr, ii, ij):
    mesh = plsc.VectorSubcoreMesh(core_axis_name="c", subcore_axis_name="s",
                                  num_cores=2, num_subcores=16)
    fn = pl.kernel(
        _gather_body,
        out_type=jax.ShapeDtypeStruct((E, 128), _f32),
        mesh=mesh,
        scratch_types=[pltpu.VMEM((2, GK), _i32),
                       pltpu.VMEM((2, GK, 64), _f32),
                       pltpu.SemaphoreType.DMA,
                       pltpu.SemaphoreType.DMA,
                       pltpu.SemaphoreType.DMA,
                       pltpu.SemaphoreType.DMA],
        compiler_params=pltpu.CompilerParams(use_tc_tiling_on_sc=False),
    )
    return fn(nl, nr, ii, ij)


def _logits_body(rel_ref, glr_ref, eg_ref, wproj_ref, bproj_ref,
                 wl_ref, wr_ref, qlr_ref, wc_ref, bc_ref, out_ref):
    d = functools.partial(jnp.dot, preferred_element_type=_f32)
    rel_emb = d(rel_ref[...], wproj_ref[...].T) + bproj_ref[...]
    rel_l = d(rel_emb, wl_ref[...][:, 64:128].T)
    rel_r = d(rel_emb, wr_ref[...][:, 64:128].T)
    eg = eg_ref[0, 0, :]
    onehot = (eg[:, None]
              == lax.broadcasted_iota(_i32, (C, B), 1)).astype(jnp.bfloat16)
    qlr = qlr_ref[...]
    qh = qlr.astype(jnp.bfloat16)
    ql = (qlr - qh.astype(_f32)).astype(jnp.bfloat16)
    q = d(onehot, qh) + d(onehot, ql)
    glr = glr_ref[...]
    pre_l = rel_l + q[:, :64] + glr[:, :64]
    pre_r = rel_r + q[:, 64:] + glr[:, 64:]
    lh = _lrelu(pre_l)
    rh = d(_lrelu(pre_r), wc_ref[...].T) + bc_ref[...]
    out_ref[0, 0, :] = jnp.sum((lh * rh).T, axis=0)


def _logits(rel, glr, eg3, wproj, bproj, wl, wr, qlr, wc, bc):
    full = lambda shape: pl.BlockSpec(shape, lambda i: (0,) * len(shape))
    return pl.pallas_call(
        _logits_body,
        grid=(GRID,),
        in_specs=[
            pl.BlockSpec((C, 256), lambda i: (i, 0)),
            pl.BlockSpec((C, 128), lambda i: (i, 0)),
            pl.BlockSpec((1, 1, C), lambda i: (i, 0, 0)),
            full((64, 256)),
            full((1, 64)),
            full((64, 256)),
            full((64, 256)),
            full((B, 128)),
            full((64, 64)),
            full((1, 64)),
        ],
        out_specs=pl.BlockSpec((1, 1, C), lambda i: (i, 0, 0)),
        out_shape=jax.ShapeDtypeStruct((GRID, 1, C), _f32),
    )(rel, glr, eg3, wproj, bproj, wl, wr, qlr, wc, bc)


def _sm_fill(ref, value):
    def body(k, _):
        ref[pl.ds(k * 16, 16)] = jnp.full((16,), value, _f32)
        return 0
    lax.fori_loop(0, NPAD // 16, body, 0)


def _softmax_body(log_hbm, ii_hbm, natt_hbm, out_hbm,
                  logv, idxv, ev, loc, mrg, tmp, stage, mst, sem):
    del sem
    c = lax.axis_index("c")
    s = lax.axis_index("s")
    base = s * EPT
    pltpu.sync_copy(log_hbm.at[pl.ds(base, EPT)], logv)
    pltpu.sync_copy(ii_hbm.at[pl.ds(base, EPT)], idxv)

    _sm_fill(loc, NEG)

    def seg_max_group(g, _):
        idx = idxv[pl.ds(g * 16, 16)]
        val = logv[pl.ds(g * 16, 16)]
        cur = plsc.load_gather(loc, [idx])
        mask = val > cur

        def cond(m):
            return jnp.any(m)

        def body(m):
            plsc.store_scatter(loc, [idx], val, mask=m)
            cur2 = plsc.load_gather(loc, [idx])
            return val > cur2

        lax.while_loop(cond, body, mask)
        return 0

    lax.fori_loop(0, EPT // 16, seg_max_group, 0)

    def merge(op):
        pltpu.sync_copy(loc, stage.at[s])
        plsc.subcore_barrier()
        col = s * NSLICE
        pltpu.sync_copy(stage.at[:, pl.ds(col, NSLICE)], tmp)

        def acc(v, _):
            sl = pl.ds(v * 16, 16)
            x = tmp[0, sl]
            for k in range(1, 16):
                x = op(x, tmp[k, sl])
            mrg[pl.ds(col + v * 16, 16)] = x
            return 0

        lax.fori_loop(0, NSLICE // 16, acc, 0)
        pltpu.sync_copy(mrg.at[pl.ds(col, NSLICE)],
                        mst.at[pl.ds(col, NSLICE)])
        plsc.subcore_barrier()
        pltpu.sync_copy(mst, mrg)

    merge(jnp.maximum)

    _sm_fill(loc, 0.0)

    def exp_group(g, _):
        sl = pl.ds(g * 16, 16)
        idx = idxv[sl]
        m = plsc.load_gather(mrg, [idx])
        e = jnp.exp(logv[sl] - m)
        ev[sl] = e
        plsc.addupdate_scatter(loc, [idx], e)
        return 0

    lax.fori_loop(0, EPT // 16, exp_group, 0)

    merge(jnp.add)

    pltpu.sync_copy(natt_hbm, loc.at[pl.ds(0, N)])

    def fac(v, _):
        sl = pl.ds(v * 16, 16)
        loc[sl] = loc[sl] / mrg[sl]
        return 0

    lax.fori_loop(0, N // 16, fac, 0)

    def out_group(g, _):
        sl = pl.ds(g * 16, 16)
        ev[sl] = ev[sl] * plsc.load_gather(loc, [idxv[sl]])
        return 0

    lax.fori_loop(0, EPT // 16, out_group, 0)

    @pl.when(c == 0)
    def _():
        pltpu.sync_copy(ev, out_hbm.at[pl.ds(base, EPT)])


def _softmax(logits, ii, natt):
    mesh = plsc.VectorSubcoreMesh(core_axis_name="c", subcore_axis_name="s",
                                  num_cores=2, num_subcores=16)
    fn = pl.kernel(
        _softmax_body,
        out_type=jax.ShapeDtypeStruct((E,), _f32),
        mesh=mesh,
        scratch_types=[
            pltpu.VMEM((EPT,), _f32),
            pltpu.VMEM((EPT,), _i32),
            pltpu.VMEM((EPT,), _f32),
            pltpu.VMEM((NPAD,), _f32),
            pltpu.VMEM((NPAD,), _f32),
            pltpu.VMEM((16, NSLICE), _f32),
            pltpu.VMEM_SHARED((16, NPAD), _f32),
            pltpu.VMEM_SHARED((NPAD,), _f32),
            pltpu.SemaphoreType.DMA,
        ],
        compiler_params=pltpu.CompilerParams(needs_layout_passes=False),
    )
    return fn(logits, ii, natt)


def kernel(attended_nodes, node_attention, selected_edges_l,
           memorized_embedding, rel_emb_l, query_src_emb, query_rel_emb,
           query_time_emb, W_proj, b_proj, W_ps, b_ps, W_pt, b_pt,
           W_left, b_left, W_right, b_right, W_center, b_center):
    del attended_nodes
    edges = selected_edges_l[-1]
    eg = edges[:, 0].astype(_i32)
    ii = edges[:, 6].astype(_i32)
    ij = edges[:, 7].astype(_i32)
    rel = rel_emb_l[-1]

    row = lambda v: v.reshape(1, -1)
    nl, nr, qlr = _prep(
        memorized_embedding, W_proj, row(b_proj), W_left, row(b_left),
        W_right, row(b_right), query_src_emb, W_ps, row(b_ps),
        query_rel_emb, query_time_emb, W_pt, row(b_pt))

    glr = _gather(nl, nr, ii, ij)

    logits3 = _logits(rel, glr, eg.reshape(GRID, 1, C), W_proj,
                      row(b_proj), W_left, W_right, qlr,
                      W_center, row(b_center))
    logits = logits3.reshape(E)

    return _softmax(logits, ii, node_attention)
